# baseline (speedup 1.0000x reference)
import functools

import numpy as np
import jax
import jax.numpy as jnp
from jax import lax
from jax.experimental import pallas as pl
from jax.experimental.pallas import tpu as pltpu
from jax.experimental.pallas import tpu_sc as plsc

_T = 1000
_TPAD = 1024
_B = 4096
_SIG = 4096

_NC, _NS, _L = 2, 16, 16
_NW = _NC * _NS
_CHUNK = _B // _NW

_R = 256


def _make_tables():
    betas = np.linspace(0.0001, 0.02, _T, dtype=np.float64)
    alpha_bar = np.cumprod(1.0 - betas)
    tab = np.zeros(2 * _TPAD, dtype=np.float64)
    tab[:_T] = np.sqrt(alpha_bar)
    tab[_TPAD:_TPAD + _T] = np.sqrt(1.0 - alpha_bar)
    return tab.astype(np.float32)


_TAB = _make_tables()

_SC_MESH = plsc.VectorSubcoreMesh(
    core_axis_name="c", subcore_axis_name="s",
    num_cores=_NC, num_subcores=_NS)


@functools.partial(
    pl.kernel,
    out_type=jax.ShapeDtypeStruct((2 * _B,), jnp.float32),
    mesh=_SC_MESH,
    scratch_types=[
        pltpu.VMEM((_CHUNK,), jnp.int32),
        pltpu.VMEM((2 * _TPAD,), jnp.float32),
        pltpu.VMEM((2 * _CHUNK,), jnp.float32),
    ],
    compiler_params=pltpu.CompilerParams(needs_layout_passes=False),
)
def _gather_coeffs(t_hbm, tab_hbm, out_hbm, idx_v, tab_v, val_v):
    wid = lax.axis_index("s") * _NC + lax.axis_index("c")
    base = wid * _CHUNK
    pltpu.sync_copy(t_hbm.at[pl.ds(base, _CHUNK)], idx_v)
    pltpu.sync_copy(tab_hbm, tab_v)

    lane = lax.iota(jnp.int32, _L)
    half = lane >> 1
    par = (lane & 1) * _TPAD

    def body(g, carry):
        idx16 = plsc.load_gather(idx_v, [g * 8 + half])
        val_v[pl.ds(g * _L, _L)] = plsc.load_gather(tab_v, [idx16 + par])
        return carry

    lax.fori_loop(0, 2 * _CHUNK // _L, body, 0)
    pltpu.sync_copy(val_v, out_hbm.at[pl.ds(2 * base, 2 * _CHUNK)])


_NROWS = _B // _NW
_CROWS = 4
_NCH = _NROWS // _CROWS
_NBUF = 4


@functools.partial(
    pl.kernel,
    out_type=jax.ShapeDtypeStruct((_B, 1, _SIG), jnp.float32),
    mesh=_SC_MESH,
    scratch_types=[
        pltpu.VMEM((_NBUF, _CROWS, 1, _SIG), jnp.float32),
        [pltpu.SemaphoreType.DMA] * _NBUF,
        [pltpu.SemaphoreType.DMA] * _NBUF,
    ],
    compiler_params=pltpu.CompilerParams(needs_layout_passes=False),
)
def _copy_noise(n_hbm, out_hbm, bufs, rsems, wsems):
    wid = lax.axis_index("s") * _NC + lax.axis_index("c")
    base = wid * _NROWS

    def rd(c):
        return pltpu.async_copy(
            n_hbm.at[pl.ds(base + c * _CROWS, _CROWS)],
            bufs.at[c % _NBUF], rsems[c % _NBUF])

    def wr(c):
        return pltpu.async_copy(
            bufs.at[c % _NBUF],
            out_hbm.at[pl.ds(base + c * _CROWS, _CROWS)], wsems[c % _NBUF])

    reads, writes = {}, {}
    for c in range(2):
        reads[c] = rd(c)
    for i in range(_NCH):
        if i + 2 < _NCH:
            if i >= 2:
                writes[i - 2].wait()
            reads[i + 2] = rd(i + 2)
        reads[i].wait()
        writes[i] = wr(i)
    writes[_NCH - 4].wait()
    writes[_NCH - 3].wait()
    writes[_NCH - 2].wait()
    writes[_NCH - 1].wait()


def _fma_body(ab_ref, x_ref, n_ref, o_ref):
    a = ab_ref[:, 0:1]
    b = ab_ref[:, 1:2]
    o_ref[...] = a * x_ref[...] + b * n_ref[...]


_fma_call = pl.pallas_call(
    _fma_body,
    grid=(_B // _R,),
    in_specs=[
        pl.BlockSpec((_R, 2), lambda i: (i, 0)),
        pl.BlockSpec((_R, None, _SIG), lambda i: (i, 0, 0)),
        pl.BlockSpec((_R, None, _SIG), lambda i: (i, 0, 0)),
    ],
    out_specs=pl.BlockSpec((_R, None, _SIG), lambda i: (i, 0, 0)),
    out_shape=jax.ShapeDtypeStruct((_B, 1, _SIG), jnp.float32),
    compiler_params=pltpu.CompilerParams(
        dimension_semantics=("arbitrary",)),
)


def kernel(x0, t, noise):
    noise_out = _copy_noise(noise)
    ab = _gather_coeffs(t, jnp.asarray(_TAB)).reshape(_B, 2)
    x_t = _fma_call(ab, x0, noise)
    return x_t, noise_out


# speedup vs baseline: 1.2592x; 1.2592x over previous
import functools

import numpy as np
import jax
import jax.numpy as jnp
from jax import lax
from jax.experimental import pallas as pl
from jax.experimental.pallas import tpu as pltpu
from jax.experimental.pallas import tpu_sc as plsc

_T = 1000
_TPAD = 1024
_B = 4096
_SIG = 4096

_NC, _NS, _L = 2, 16, 16
_NW = _NC * _NS
_CHUNK = _B // _NW

_R = 256


def _make_tables():
    betas = np.linspace(0.0001, 0.02, _T, dtype=np.float64)
    alpha_bar = np.cumprod(1.0 - betas)
    tab = np.zeros(2 * _TPAD, dtype=np.float64)
    tab[:_T] = np.sqrt(alpha_bar)
    tab[_TPAD:_TPAD + _T] = np.sqrt(1.0 - alpha_bar)
    return tab.astype(np.float32)


_TAB = _make_tables()

_SC_MESH = plsc.VectorSubcoreMesh(
    core_axis_name="c", subcore_axis_name="s",
    num_cores=_NC, num_subcores=_NS)


@functools.partial(
    pl.kernel,
    out_type=jax.ShapeDtypeStruct((2, 1, _B), jnp.float32),
    mesh=_SC_MESH,
    scratch_types=[
        pltpu.VMEM((_CHUNK,), jnp.int32),
        pltpu.VMEM((2 * _TPAD,), jnp.float32),
        pltpu.VMEM((_CHUNK,), jnp.float32),
        pltpu.VMEM((_CHUNK,), jnp.float32),
    ],
    compiler_params=pltpu.CompilerParams(needs_layout_passes=False),
)
def _gather_coeffs(t_hbm, tab_hbm, out_hbm, idx_v, tab_v, va_v, vb_v):
    wid = lax.axis_index("s") * _NC + lax.axis_index("c")
    base = wid * _CHUNK
    pltpu.sync_copy(t_hbm.at[pl.ds(base, _CHUNK)], idx_v)
    pltpu.sync_copy(tab_hbm, tab_v)

    def body(i, carry):
        idx16 = idx_v[pl.ds(i * _L, _L)]
        va_v[pl.ds(i * _L, _L)] = plsc.load_gather(tab_v, [idx16])
        vb_v[pl.ds(i * _L, _L)] = plsc.load_gather(tab_v, [idx16 + _TPAD])
        return carry

    lax.fori_loop(0, _CHUNK // _L, body, 0)
    pltpu.sync_copy(va_v, out_hbm.at[0, 0, pl.ds(base, _CHUNK)])
    pltpu.sync_copy(vb_v, out_hbm.at[1, 0, pl.ds(base, _CHUNK)])


def _fma_body(a_ref, b_ref, x_ref, n_ref, o_ref, ncopy_ref):
    n = n_ref[...]
    a_col = jnp.transpose(a_ref[...], (1, 0))
    b_col = jnp.transpose(b_ref[...], (1, 0))
    o_ref[...] = a_col * x_ref[...] + b_col * n
    ncopy_ref[...] = n


_fma_call = pl.pallas_call(
    _fma_body,
    grid=(_B // _R,),
    in_specs=[
        pl.BlockSpec((None, 1, _R), lambda i: (0, 0, i)),
        pl.BlockSpec((None, 1, _R), lambda i: (1, 0, i)),
        pl.BlockSpec((_R, None, _SIG), lambda i: (i, 0, 0)),
        pl.BlockSpec((_R, None, _SIG), lambda i: (i, 0, 0)),
    ],
    out_specs=[
        pl.BlockSpec((_R, None, _SIG), lambda i: (i, 0, 0)),
        pl.BlockSpec((_R, None, _SIG), lambda i: (i, 0, 0)),
    ],
    out_shape=[
        jax.ShapeDtypeStruct((_B, 1, _SIG), jnp.float32),
        jax.ShapeDtypeStruct((_B, 1, _SIG), jnp.float32),
    ],
    compiler_params=pltpu.CompilerParams(
        dimension_semantics=("arbitrary",)),
)


def kernel(x0, t, noise):
    ab = _gather_coeffs(t, jnp.asarray(_TAB))
    x_t, noise_out = _fma_call(ab, ab, x0, noise)
    return x_t, noise_out
